# baseline (speedup 1.0000x reference)
import functools

import jax
import jax.numpy as jnp
from jax import lax
from jax.experimental import pallas as pl
from jax.experimental.pallas import tpu as pltpu
from jax.experimental.pallas import tpu_sc as plsc

H = 128
ATOM_D = 128
BOND_D = 144
N_ATOMS = 10001
N_BONDS = 320000
MAX_NB = 32
N_MOLS = 100
MOL_SIZE = 100
DEPTH = 3

NA_PAD = 10240
NB_PAD = 327680


NC, NS = 2, 16
NW = NC * NS


def _sc_mesh():
    return plsc.VectorSubcoreMesh(core_axis_name="c", subcore_axis_name="s")


def _sc_agg(mb, a2b_r):
    NBUF = 4
    B0A, B1A = 128, 32

    @functools.partial(
        pl.kernel,
        out_type=jax.ShapeDtypeStruct((NA_PAD, H), jnp.float32),
        mesh=_sc_mesh(),
        scratch_types=[
            pltpu.VMEM((B0A, 128), jnp.int32),
            [pltpu.VMEM((128, H), jnp.float32)] * NBUF,
            pltpu.VMEM((64, H), jnp.float32),
            [pltpu.SemaphoreType.DMA] * NBUF,
        ],
    )
    def k(mb_hbm, a2b_hbm, out_hbm, idx_v, rows, obuf, sems):
        c = lax.axis_index("c")
        s = lax.axis_index("s")
        nblk = jnp.where(c == 0, B0A, B1A)
        blk0 = jnp.where(c == 0, s * B0A, NS * B0A + s * B1A)
        pltpu.sync_copy(a2b_hbm.at[pl.ds(pl.multiple_of(blk0, 8), B0A)], idx_v)

        def compute_block(j, rbuf, ob, off):
            def atom_body(a, _):
                base = a * 32
                for ch in range(8):
                    sl = pl.ds(ch * 16, 16)
                    sv = []
                    m = None
                    for q in range(8):
                        x0 = rbuf[base + q, sl]
                        x1 = rbuf[base + q + 8, sl]
                        x2 = rbuf[base + q + 16, sl]
                        x3 = rbuf[base + q + 24, sl]
                        sv.append(((x0 + x1) + x2) + x3)
                        mm = jnp.maximum(jnp.maximum(x0, x1), jnp.maximum(x2, x3))
                        m = mm if m is None else jnp.maximum(m, mm)
                    t = [sv[q] + sv[q + 4] for q in range(4)]
                    u = [t[q] + t[q + 2] for q in range(2)]
                    ob[off + a, sl] = (u[0] + u[1]) * m
                return 0

            lax.fori_loop(0, 4, atom_body, 0)

        for b in range(NBUF):
            pltpu.async_copy(mb_hbm.at[idx_v.at[b]], rows[b], sems[b])

        def loop(jj, _):
            for b in range(NBUF):
                j = jj * NBUF + b
                pltpu.make_async_copy(mb_hbm.at[idx_v.at[j]], rows[b], sems[b]).wait()
                compute_block(j, rows[b], obuf, (j % 16) * 4)

                @pl.when(j + NBUF < nblk)
                def _():
                    pltpu.async_copy(mb_hbm.at[idx_v.at[j + NBUF]], rows[b], sems[b])

                @pl.when(j % 16 == 15)
                def _():
                    pltpu.sync_copy(
                        obuf,
                        out_hbm.at[pl.ds(pl.multiple_of((blk0 + j - 15) * 4, 8), 64)],
                    )
            return 0

        lax.fori_loop(0, nblk // NBUF, loop, 0)

    return k(mb, a2b_r)


def _sc_pre(ma, mb, b2a_r, b2revb_r):
    BLK = 64
    NBUF = 4
    B0P, B1P = 192, 128
    HALF = B0P // 2

    @functools.partial(
        pl.kernel,
        out_type=jax.ShapeDtypeStruct((NB_PAD, H), jnp.float32),
        mesh=_sc_mesh(),
        scratch_types=[
            pltpu.VMEM((HALF, BLK), jnp.int32),
            pltpu.VMEM((HALF, BLK), jnp.int32),
            [pltpu.VMEM((BLK, H), jnp.float32)] * NBUF,
            [pltpu.VMEM((BLK, H), jnp.float32)] * NBUF,
            [pltpu.VMEM((BLK, H), jnp.float32)] * 2,
            [pltpu.SemaphoreType.DMA] * NBUF,
            [pltpu.SemaphoreType.DMA] * NBUF,
            [pltpu.SemaphoreType.DMA] * 2,
        ],
    )
    def k(ma_hbm, mb_hbm, b2a_hbm, b2revb_hbm, out_hbm,
          ia_v, ib_v, abufs, bbufs, obufs, asems, bsems, osems):
        c = lax.axis_index("c")
        s = lax.axis_index("s")
        nblk = jnp.where(c == 0, B0P, B1P)
        blk0 = jnp.where(c == 0, s * B0P, NS * B0P + s * B1P)

        def start(jl, b):
            pltpu.async_copy(ma_hbm.at[ia_v.at[jl]], abufs[b], asems[b])
            pltpu.async_copy(mb_hbm.at[ib_v.at[jl]], bbufs[b], bsems[b])

        def phase(off, halfn):
            pltpu.sync_copy(
                b2a_hbm.at[pl.ds(pl.multiple_of(blk0 + off, 8), HALF)], ia_v)
            pltpu.sync_copy(
                b2revb_hbm.at[pl.ds(pl.multiple_of(blk0 + off, 8), HALF)], ib_v)
            for b in range(NBUF):
                start(b, b)

            def loop(jj, _):
                for b in range(NBUF):
                    jl = jj * NBUF + b
                    jg = off + jl
                    ob = b % 2
                    pltpu.make_async_copy(ma_hbm.at[ia_v.at[jl]], abufs[b], asems[b]).wait()
                    pltpu.make_async_copy(mb_hbm.at[ib_v.at[jl]], bbufs[b], bsems[b]).wait()

                    @pl.when(jg >= 2)
                    def _():
                        pltpu.make_async_copy(
                            obufs[ob],
                            out_hbm.at[pl.ds(pl.multiple_of((blk0 + jg - 2) * BLK, 8), BLK)],
                            osems[ob],
                        ).wait()

                    def row_body(r, _):
                        for ch in range(8):
                            sl = pl.ds(ch * 16, 16)
                            obufs[ob][r, sl] = abufs[b][r, sl] - bbufs[b][r, sl]
                        return 0

                    lax.fori_loop(0, BLK, row_body, 0)

                    @pl.when(jl + NBUF < halfn)
                    def _():
                        start(jl + NBUF, b)

                    pltpu.async_copy(
                        obufs[ob],
                        out_hbm.at[pl.ds(pl.multiple_of((blk0 + jg) * BLK, 8), BLK)],
                        osems[ob],
                    )
                return 0

            lax.fori_loop(0, halfn // NBUF, loop, 0)

        phase(0, nblk // 2)
        phase(nblk // 2, nblk // 2)
        for ob in range(2):
            pltpu.make_async_copy(
                obufs[ob],
                out_hbm.at[pl.ds(pl.multiple_of((blk0 + nblk - 2 + ob) * BLK, 8), BLK)],
                osems[ob],
            ).wait()

    return k(ma, mb, b2a_r, b2revb_r)


def _pad_rows(x, n):
    return jnp.pad(x, ((0, n - x.shape[0]),) + ((0, 0),) * (x.ndim - 1))


def _mm(xws, adds=(), bias=None, relu=False, pre=None, block=512, rows=None):
    n = xws[0][0].shape[0]
    rows = n if rows is None else rows
    grid = rows // block
    hout = xws[0][1].shape[1]
    in_specs = []
    args = []
    for x, w in xws:
        in_specs.append(pl.BlockSpec((block, x.shape[1]), lambda i: (i, 0)))
        in_specs.append(pl.BlockSpec(w.shape, lambda i: (0, 0)))
        args += [x, w]
    for a in adds:
        in_specs.append(pl.BlockSpec((block, a.shape[1]), lambda i: (i, 0)))
        args.append(a)
    if bias is not None:
        b2 = bias.reshape(1, hout)
        in_specs.append(pl.BlockSpec((1, hout), lambda i: (0, 0)))
        args.append(b2)
    if pre is not None:
        p2 = pre.reshape(1, xws[0][0].shape[1])
        in_specs.append(pl.BlockSpec((1, p2.shape[1]), lambda i: (0, 0)))
        args.append(p2)
    nxw = len(xws)
    nadd = len(adds)

    def body(*refs):
        out_ref = refs[-1]
        k = 2 * nxw + nadd
        b_ref = refs[k] if bias is not None else None
        p_ref = refs[k + (1 if bias is not None else 0)] if pre is not None else None
        acc = None
        for i in range(nxw):
            xv = refs[2 * i][...]
            if p_ref is not None:
                xv = jnp.maximum(xv + p_ref[...], 0.0)
            t = jnp.dot(xv, refs[2 * i + 1][...], preferred_element_type=jnp.float32)
            acc = t if acc is None else acc + t
        for i in range(nadd):
            acc = acc + refs[2 * nxw + i][...]
        if b_ref is not None:
            acc = acc + b_ref[...]
        if relu:
            acc = jnp.maximum(acc, 0.0)
        out_ref[...] = acc

    return pl.pallas_call(
        body,
        grid=(grid,),
        in_specs=in_specs,
        out_specs=pl.BlockSpec((block, hout), lambda i: (i, 0)),
        out_shape=jax.ShapeDtypeStruct((n, hout), jnp.float32),
    )(*args)


def _h0_max(nm3):
    def body(x_ref, o_ref):
        o_ref[...] = jnp.max(x_ref[...], axis=1)

    return pl.pallas_call(
        body,
        out_shape=jax.ShapeDtypeStruct((N_MOLS, H), jnp.float32),
    )(nm3)


def _gru_dir(gi3, h0, whhT, bhh, reverse):
    T, Bp = gi3.shape[0], gi3.shape[1]

    def body(gi_ref, h0_ref, w_ref, b_ref, out_ref, h_ref):
        t = pl.program_id(0)

        @pl.when(t == 0)
        def _():
            h_ref[...] = h0_ref[...]

        h = h_ref[...]
        gh = jnp.dot(h, w_ref[...], preferred_element_type=jnp.float32) + b_ref[...]
        gi = gi_ref[0]
        r = jax.nn.sigmoid(gi[:, :H] + gh[:, :H])
        z = jax.nn.sigmoid(gi[:, H:2 * H] + gh[:, H:2 * H])
        nn = jnp.tanh(gi[:, 2 * H:] + r * gh[:, 2 * H:])
        hn = (1.0 - z) * nn + z * h
        h_ref[...] = hn
        out_ref[0] = hn

    if reverse:
        idx = lambda t: (T - 1 - t, 0, 0)
    else:
        idx = lambda t: (t, 0, 0)
    return pl.pallas_call(
        body,
        grid=(T,),
        in_specs=[
            pl.BlockSpec((1, Bp, 3 * H), idx),
            pl.BlockSpec((Bp, H), lambda t: (0, 0)),
            pl.BlockSpec((H, 3 * H), lambda t: (0, 0)),
            pl.BlockSpec((1, 3 * H), lambda t: (0, 0)),
        ],
        out_specs=pl.BlockSpec((1, Bp, H), idx),
        out_shape=jax.ShapeDtypeStruct((T, Bp, H), jnp.float32),
        scratch_shapes=[pltpu.VMEM((Bp, H), jnp.float32)],
    )(gi3, h0, whhT, bhh.reshape(1, 3 * H))


def kernel(f_atoms, f_bonds, a2b, b2a, b2revb, a_scope, W_i_atom, W_i_bond,
           W_h_0, W_h_1, lr_W, W_o, b_o, gru_bias, W_ih_f, W_hh_f, b_ih_f,
           b_hh_f, W_ih_r, W_hh_r, b_ih_r, b_hh_r):
    a2b_r = _pad_rows(_pad_rows(a2b.astype(jnp.int32), NA_PAD).reshape(-1, 128), 2688)
    b2a_r = _pad_rows(
        jnp.pad(b2a.astype(jnp.int32), (0, NB_PAD - N_BONDS)).reshape(-1, 64), 5280)
    b2revb_r = _pad_rows(
        jnp.pad(b2revb.astype(jnp.int32), (0, NB_PAD - N_BONDS)).reshape(-1, 64), 5280)

    fa_p = _pad_rows(f_atoms, NA_PAD)
    ia = _mm([(fa_p, W_i_atom.T)], relu=True)
    ib = _mm([(f_bonds, W_i_bond.T)], relu=True)

    ma = ia
    mb = ib
    Whs = [W_h_0, W_h_1]
    for d in range(DEPTH - 1):
        ma = ma + _sc_agg(mb, a2b_r)
        pre = _sc_pre(ma, mb, b2a_r, b2revb_r)
        mb = _mm([(pre, Whs[d].T)], adds=(ib,), relu=True, rows=N_BONDS)

    aggf = _sc_agg(mb, a2b_r)

    cat = jnp.concatenate([aggf, ma, ia], axis=1)
    node = _mm([(cat, lr_W.T)])

    node_seq = node[1:1 + N_MOLS * MOL_SIZE]
    nm3 = node_seq.reshape(N_MOLS, MOL_SIZE, H)
    h0 = _h0_max(nm3)
    Bp = 128
    h0p = _pad_rows(h0, Bp)
    xs_t = jnp.pad(nm3.transpose(1, 0, 2), ((0, 0), (0, Bp - N_MOLS), (0, 0)))
    xs_flat = xs_t.reshape(MOL_SIZE * Bp, H)
    gif = _mm([(xs_flat, W_ih_f.T)], bias=b_ih_f, pre=gru_bias).reshape(MOL_SIZE, Bp, 3 * H)
    gib = _mm([(xs_flat, W_ih_r.T)], bias=b_ih_r, pre=gru_bias).reshape(MOL_SIZE, Bp, 3 * H)
    fwd = _gru_dir(gif, h0p, W_hh_f.T, b_hh_f, reverse=False)
    bwd = _gru_dir(gib, h0p, W_hh_r.T, b_hh_r, reverse=True)
    fwd_mol = fwd[:, :N_MOLS].transpose(1, 0, 2).reshape(N_MOLS * MOL_SIZE, H)
    bwd_mol = bwd[:, :N_MOLS].transpose(1, 0, 2).reshape(N_MOLS * MOL_SIZE, H)

    msg0 = jnp.maximum(node[0:1] + gru_bias[None, :], 0.0)
    fwd_full = _pad_rows(jnp.concatenate([msg0, fwd_mol], axis=0), NA_PAD)
    bwd_full = _pad_rows(jnp.concatenate([msg0, bwd_mol], axis=0), NA_PAD)

    msg = jnp.concatenate([fwd_full, bwd_full], axis=1)
    out = _mm([(msg, W_o.T)], bias=b_o, relu=True)
    return out[:N_ATOMS]


# speedup vs baseline: 1.0101x; 1.0101x over previous
import functools

import jax
import jax.numpy as jnp
from jax import lax
from jax.experimental import pallas as pl
from jax.experimental.pallas import tpu as pltpu
from jax.experimental.pallas import tpu_sc as plsc

H = 128
ATOM_D = 128
BOND_D = 144
N_ATOMS = 10001
N_BONDS = 320000
MAX_NB = 32
N_MOLS = 100
MOL_SIZE = 100
DEPTH = 3

NA_PAD = 10240
NB_PAD = 327680


NC, NS = 2, 16
NW = NC * NS


def _sc_mesh():
    return plsc.VectorSubcoreMesh(core_axis_name="c", subcore_axis_name="s")


def _sc_agg(mb, a2b_r):
    NBUF = 4
    B0A, B1A = 144, 16

    @functools.partial(
        pl.kernel,
        out_type=jax.ShapeDtypeStruct((NA_PAD, H), jnp.float32),
        mesh=_sc_mesh(),
        scratch_types=[
            pltpu.VMEM((B0A, 128), jnp.int32),
            [pltpu.VMEM((128, H), jnp.float32)] * NBUF,
            pltpu.VMEM((64, H), jnp.float32),
            [pltpu.SemaphoreType.DMA] * NBUF,
        ],
    )
    def k(mb_hbm, a2b_hbm, out_hbm, idx_v, rows, obuf, sems):
        c = lax.axis_index("c")
        s = lax.axis_index("s")
        nblk = jnp.where(c == 0, B0A, B1A)
        blk0 = jnp.where(c == 0, s * B0A, NS * B0A + s * B1A)
        pltpu.sync_copy(a2b_hbm.at[pl.ds(pl.multiple_of(blk0, 8), B0A)], idx_v)

        def compute_block(j, rbuf, ob, off):
            def atom_body(a, _):
                base = a * 32
                for ch in range(8):
                    sl = pl.ds(ch * 16, 16)
                    sv = []
                    m = None
                    for q in range(8):
                        x0 = rbuf[base + q, sl]
                        x1 = rbuf[base + q + 8, sl]
                        x2 = rbuf[base + q + 16, sl]
                        x3 = rbuf[base + q + 24, sl]
                        sv.append(((x0 + x1) + x2) + x3)
                        mm = jnp.maximum(jnp.maximum(x0, x1), jnp.maximum(x2, x3))
                        m = mm if m is None else jnp.maximum(m, mm)
                    t = [sv[q] + sv[q + 4] for q in range(4)]
                    u = [t[q] + t[q + 2] for q in range(2)]
                    ob[off + a, sl] = (u[0] + u[1]) * m
                return 0

            lax.fori_loop(0, 4, atom_body, 0)

        for b in range(NBUF):
            pltpu.async_copy(mb_hbm.at[idx_v.at[b]], rows[b], sems[b])

        def loop(jj, _):
            for b in range(NBUF):
                j = jj * NBUF + b
                pltpu.make_async_copy(mb_hbm.at[idx_v.at[j]], rows[b], sems[b]).wait()
                compute_block(j, rows[b], obuf, (j % 16) * 4)

                @pl.when(j + NBUF < nblk)
                def _():
                    pltpu.async_copy(mb_hbm.at[idx_v.at[j + NBUF]], rows[b], sems[b])

                @pl.when(j % 16 == 15)
                def _():
                    pltpu.sync_copy(
                        obuf,
                        out_hbm.at[pl.ds(pl.multiple_of((blk0 + j - 15) * 4, 8), 64)],
                    )
            return 0

        lax.fori_loop(0, nblk // NBUF, loop, 0)

    return k(mb, a2b_r)


def _sc_pre(ma, mb, b2a_r, b2revb_r):
    BLK = 64
    NBUF = 4
    B0P, B1P = 240, 80
    HALF = B0P // 2

    @functools.partial(
        pl.kernel,
        out_type=jax.ShapeDtypeStruct((NB_PAD, H), jnp.float32),
        mesh=_sc_mesh(),
        scratch_types=[
            pltpu.VMEM((HALF, BLK), jnp.int32),
            pltpu.VMEM((HALF, BLK), jnp.int32),
            [pltpu.VMEM((BLK, H), jnp.float32)] * NBUF,
            [pltpu.VMEM((BLK, H), jnp.float32)] * NBUF,
            [pltpu.VMEM((BLK, H), jnp.float32)] * 2,
            [pltpu.SemaphoreType.DMA] * NBUF,
            [pltpu.SemaphoreType.DMA] * NBUF,
            [pltpu.SemaphoreType.DMA] * 2,
        ],
    )
    def k(ma_hbm, mb_hbm, b2a_hbm, b2revb_hbm, out_hbm,
          ia_v, ib_v, abufs, bbufs, obufs, asems, bsems, osems):
        c = lax.axis_index("c")
        s = lax.axis_index("s")
        nblk = jnp.where(c == 0, B0P, B1P)
        blk0 = jnp.where(c == 0, s * B0P, NS * B0P + s * B1P)

        def start(jl, b):
            pltpu.async_copy(ma_hbm.at[ia_v.at[jl]], abufs[b], asems[b])
            pltpu.async_copy(mb_hbm.at[ib_v.at[jl]], bbufs[b], bsems[b])

        def phase(off, halfn):
            pltpu.sync_copy(
                b2a_hbm.at[pl.ds(pl.multiple_of(blk0 + off, 8), HALF)], ia_v)
            pltpu.sync_copy(
                b2revb_hbm.at[pl.ds(pl.multiple_of(blk0 + off, 8), HALF)], ib_v)
            for b in range(NBUF):
                start(b, b)

            def loop(jj, _):
                for b in range(NBUF):
                    jl = jj * NBUF + b
                    jg = off + jl
                    ob = b % 2
                    pltpu.make_async_copy(ma_hbm.at[ia_v.at[jl]], abufs[b], asems[b]).wait()
                    pltpu.make_async_copy(mb_hbm.at[ib_v.at[jl]], bbufs[b], bsems[b]).wait()

                    @pl.when(jg >= 2)
                    def _():
                        pltpu.make_async_copy(
                            obufs[ob],
                            out_hbm.at[pl.ds(pl.multiple_of((blk0 + jg - 2) * BLK, 8), BLK)],
                            osems[ob],
                        ).wait()

                    def row_body(r, _):
                        for ch in range(8):
                            sl = pl.ds(ch * 16, 16)
                            obufs[ob][r, sl] = abufs[b][r, sl] - bbufs[b][r, sl]
                        return 0

                    lax.fori_loop(0, BLK, row_body, 0)

                    @pl.when(jl + NBUF < halfn)
                    def _():
                        start(jl + NBUF, b)

                    pltpu.async_copy(
                        obufs[ob],
                        out_hbm.at[pl.ds(pl.multiple_of((blk0 + jg) * BLK, 8), BLK)],
                        osems[ob],
                    )
                return 0

            lax.fori_loop(0, halfn // NBUF, loop, 0)

        phase(0, nblk // 2)
        phase(nblk // 2, nblk // 2)
        for ob in range(2):
            pltpu.make_async_copy(
                obufs[ob],
                out_hbm.at[pl.ds(pl.multiple_of((blk0 + nblk - 2 + ob) * BLK, 8), BLK)],
                osems[ob],
            ).wait()

    return k(ma, mb, b2a_r, b2revb_r)


def _pad_rows(x, n):
    return jnp.pad(x, ((0, n - x.shape[0]),) + ((0, 0),) * (x.ndim - 1))


def _mm(xws, adds=(), bias=None, relu=False, pre=None, block=512, rows=None):
    n = xws[0][0].shape[0]
    rows = n if rows is None else rows
    grid = rows // block
    hout = xws[0][1].shape[1]
    in_specs = []
    args = []
    for x, w in xws:
        in_specs.append(pl.BlockSpec((block, x.shape[1]), lambda i: (i, 0)))
        in_specs.append(pl.BlockSpec(w.shape, lambda i: (0, 0)))
        args += [x, w]
    for a in adds:
        in_specs.append(pl.BlockSpec((block, a.shape[1]), lambda i: (i, 0)))
        args.append(a)
    if bias is not None:
        b2 = bias.reshape(1, hout)
        in_specs.append(pl.BlockSpec((1, hout), lambda i: (0, 0)))
        args.append(b2)
    if pre is not None:
        p2 = pre.reshape(1, xws[0][0].shape[1])
        in_specs.append(pl.BlockSpec((1, p2.shape[1]), lambda i: (0, 0)))
        args.append(p2)
    nxw = len(xws)
    nadd = len(adds)

    def body(*refs):
        out_ref = refs[-1]
        k = 2 * nxw + nadd
        b_ref = refs[k] if bias is not None else None
        p_ref = refs[k + (1 if bias is not None else 0)] if pre is not None else None
        acc = None
        for i in range(nxw):
            xv = refs[2 * i][...]
            if p_ref is not None:
                xv = jnp.maximum(xv + p_ref[...], 0.0)
            t = jnp.dot(xv, refs[2 * i + 1][...], preferred_element_type=jnp.float32)
            acc = t if acc is None else acc + t
        for i in range(nadd):
            acc = acc + refs[2 * nxw + i][...]
        if b_ref is not None:
            acc = acc + b_ref[...]
        if relu:
            acc = jnp.maximum(acc, 0.0)
        out_ref[...] = acc

    return pl.pallas_call(
        body,
        grid=(grid,),
        in_specs=in_specs,
        out_specs=pl.BlockSpec((block, hout), lambda i: (i, 0)),
        out_shape=jax.ShapeDtypeStruct((n, hout), jnp.float32),
    )(*args)


def _h0_max(nm3):
    def body(x_ref, o_ref):
        o_ref[...] = jnp.max(x_ref[...], axis=1)

    return pl.pallas_call(
        body,
        out_shape=jax.ShapeDtypeStruct((N_MOLS, H), jnp.float32),
    )(nm3)


def _gru_dir(gi3, h0, whhT, bhh, reverse):
    T, Bp = gi3.shape[0], gi3.shape[1]

    def body(gi_ref, h0_ref, w_ref, b_ref, out_ref, h_ref):
        t = pl.program_id(0)

        @pl.when(t == 0)
        def _():
            h_ref[...] = h0_ref[...]

        h = h_ref[...]
        gh = jnp.dot(h, w_ref[...], preferred_element_type=jnp.float32) + b_ref[...]
        gi = gi_ref[0]
        r = jax.nn.sigmoid(gi[:, :H] + gh[:, :H])
        z = jax.nn.sigmoid(gi[:, H:2 * H] + gh[:, H:2 * H])
        nn = jnp.tanh(gi[:, 2 * H:] + r * gh[:, 2 * H:])
        hn = (1.0 - z) * nn + z * h
        h_ref[...] = hn
        out_ref[0] = hn

    if reverse:
        idx = lambda t: (T - 1 - t, 0, 0)
    else:
        idx = lambda t: (t, 0, 0)
    return pl.pallas_call(
        body,
        grid=(T,),
        in_specs=[
            pl.BlockSpec((1, Bp, 3 * H), idx),
            pl.BlockSpec((Bp, H), lambda t: (0, 0)),
            pl.BlockSpec((H, 3 * H), lambda t: (0, 0)),
            pl.BlockSpec((1, 3 * H), lambda t: (0, 0)),
        ],
        out_specs=pl.BlockSpec((1, Bp, H), idx),
        out_shape=jax.ShapeDtypeStruct((T, Bp, H), jnp.float32),
        scratch_shapes=[pltpu.VMEM((Bp, H), jnp.float32)],
    )(gi3, h0, whhT, bhh.reshape(1, 3 * H))


def kernel(f_atoms, f_bonds, a2b, b2a, b2revb, a_scope, W_i_atom, W_i_bond,
           W_h_0, W_h_1, lr_W, W_o, b_o, gru_bias, W_ih_f, W_hh_f, b_ih_f,
           b_hh_f, W_ih_r, W_hh_r, b_ih_r, b_hh_r):
    a2b_r = _pad_rows(_pad_rows(a2b.astype(jnp.int32), NA_PAD).reshape(-1, 128), 2688)
    b2a_r = _pad_rows(
        jnp.pad(b2a.astype(jnp.int32), (0, NB_PAD - N_BONDS)).reshape(-1, 64), 5280)
    b2revb_r = _pad_rows(
        jnp.pad(b2revb.astype(jnp.int32), (0, NB_PAD - N_BONDS)).reshape(-1, 64), 5280)

    fa_p = _pad_rows(f_atoms, NA_PAD)
    ia = _mm([(fa_p, W_i_atom.T)], relu=True)
    ib = _mm([(f_bonds, W_i_bond.T)], relu=True)

    ma = ia
    mb = ib
    Whs = [W_h_0, W_h_1]
    for d in range(DEPTH - 1):
        ma = ma + _sc_agg(mb, a2b_r)
        pre = _sc_pre(ma, mb, b2a_r, b2revb_r)
        mb = _mm([(pre, Whs[d].T)], adds=(ib,), relu=True, rows=N_BONDS)

    aggf = _sc_agg(mb, a2b_r)

    cat = jnp.concatenate([aggf, ma, ia], axis=1)
    node = _mm([(cat, lr_W.T)])

    node_seq = node[1:1 + N_MOLS * MOL_SIZE]
    nm3 = node_seq.reshape(N_MOLS, MOL_SIZE, H)
    h0 = _h0_max(nm3)
    Bp = 128
    h0p = _pad_rows(h0, Bp)
    xs_t = jnp.pad(nm3.transpose(1, 0, 2), ((0, 0), (0, Bp - N_MOLS), (0, 0)))
    xs_flat = xs_t.reshape(MOL_SIZE * Bp, H)
    gif = _mm([(xs_flat, W_ih_f.T)], bias=b_ih_f, pre=gru_bias).reshape(MOL_SIZE, Bp, 3 * H)
    gib = _mm([(xs_flat, W_ih_r.T)], bias=b_ih_r, pre=gru_bias).reshape(MOL_SIZE, Bp, 3 * H)
    fwd = _gru_dir(gif, h0p, W_hh_f.T, b_hh_f, reverse=False)
    bwd = _gru_dir(gib, h0p, W_hh_r.T, b_hh_r, reverse=True)
    fwd_mol = fwd[:, :N_MOLS].transpose(1, 0, 2).reshape(N_MOLS * MOL_SIZE, H)
    bwd_mol = bwd[:, :N_MOLS].transpose(1, 0, 2).reshape(N_MOLS * MOL_SIZE, H)

    msg0 = jnp.maximum(node[0:1] + gru_bias[None, :], 0.0)
    fwd_full = _pad_rows(jnp.concatenate([msg0, fwd_mol], axis=0), NA_PAD)
    bwd_full = _pad_rows(jnp.concatenate([msg0, bwd_mol], axis=0), NA_PAD)

    msg = jnp.concatenate([fwd_full, bwd_full], axis=1)
    out = _mm([(msg, W_o.T)], bias=b_o, relu=True)
    return out[:N_ATOMS]
